# baseline (speedup 1.0000x reference)
import functools

import jax
import jax.numpy as jnp
from jax import lax
from jax.experimental import pallas as pl
from jax.experimental.pallas import tpu as pltpu
from jax.experimental.pallas import tpu_sc as plsc

_BSL = 512
_NT = 50
_D = 64


def _transpose_to_tiles(g_ref, tr_ref, iota):
    def irow(i, carry):
        for r in range(8):
            col_vec = jnp.zeros((16,), jnp.int32) + (8 * i + r)
            for j in range(4):
                for cs in range(8):
                    row_vec = iota + (128 * j + 16 * cs)
                    vals = plsc.load_gather(g_ref, [row_vec, col_vec])
                    tr_ref[i, j, r, pl.ds(16 * cs, 16)] = vals
        return carry

    lax.fori_loop(0, 8, irow, 0)


def _emb_kernel(num_cores, idx_hbm, table_hbm, out_hbm,
                idx_v, g_a, g_b, tr, sg_a, sg_b, ss):
    wid = lax.axis_index("s") * num_cores + lax.axis_index("c")
    b0 = wid * _BSL
    cblk = wid * (_BSL // 128)
    iota = lax.iota(jnp.int32, 16)

    pltpu.sync_copy(idx_hbm.at[:, pl.ds(b0, _BSL)], idx_v)

    def start_gather(t, g, sem):
        pltpu.async_copy(table_hbm.at[idx_v.at[t]], g, sem)

    def wait_gather(g, sem):
        pltpu.make_async_copy(table_hbm.at[idx_v.at[0]], g, sem).wait()

    def wait_store():
        pltpu.make_async_copy(
            tr, out_hbm.at[pl.ds(0, 8), pl.ds(cblk, 4)], ss).wait()

    def step(t, g, sem, first):
        wait_gather(g, sem)
        if not first:
            wait_store()
        _transpose_to_tiles(g, tr, iota)
        pltpu.async_copy(tr, out_hbm.at[pl.ds(8 * t, 8), pl.ds(cblk, 4)], ss)
        start_gather(jnp.minimum(t + 2, _NT - 1), g, sem)

    start_gather(0, g_a, sg_a)
    start_gather(1, g_b, sg_b)
    step(0, g_a, sg_a, True)
    step(1, g_b, sg_b, False)

    def body(k, carry):
        step(2 + 2 * k, g_a, sg_a, False)
        step(3 + 2 * k, g_b, sg_b, False)
        return carry

    lax.fori_loop(0, (_NT - 2) // 2, body, 0)
    wait_store()
    wait_gather(g_a, sg_a)
    wait_gather(g_b, sg_b)


def kernel(input_ids, table):
    B, S = input_ids.shape
    V, D = table.shape
    idx2d = input_ids.T.astype(jnp.int32)

    info = plsc.get_sparse_core_info()
    nw = info.num_cores * info.num_subcores
    assert B == nw * _BSL and S == _NT and D == _D

    mesh = plsc.VectorSubcoreMesh(core_axis_name="c", subcore_axis_name="s")
    emb = functools.partial(
        pl.kernel,
        mesh=mesh,
        out_type=jax.ShapeDtypeStruct((S * 8, B // 128, 8, 128), jnp.float32),
        scratch_types=[
            pltpu.VMEM((_NT, _BSL), jnp.int32),
            pltpu.VMEM((_BSL, _D), jnp.float32),
            pltpu.VMEM((_BSL, _D), jnp.float32),
            pltpu.VMEM((8, _BSL // 128, 8, 128), jnp.float32),
            pltpu.SemaphoreType.DMA,
            pltpu.SemaphoreType.DMA,
            pltpu.SemaphoreType.DMA,
        ],
        compiler_params=pltpu.CompilerParams(use_tc_tiling_on_sc=False,
                                             needs_layout_passes=False),
    )(functools.partial(_emb_kernel, info.num_cores))

    i1 = emb(idx2d, table)
    out = (i1.reshape(S, 8, B // 128, 8, 128)
             .transpose(2, 4, 0, 1, 3)
             .reshape(B, S, D))
    return out


# speedup vs baseline: 1.2843x; 1.2843x over previous
import functools

import jax
import jax.numpy as jnp
from jax import lax
from jax.experimental import pallas as pl
from jax.experimental.pallas import tpu as pltpu
from jax.experimental.pallas import tpu_sc as plsc

_BSL = 512
_NT = 50
_D = 64


def _transpose_to_tiles(g_ref, tr_ref, iota):
    @plsc.parallel_loop(0, 2048, unroll=8)
    def body(k):
        i = k >> 8
        j = (k >> 6) & 3
        r = (k >> 3) & 7
        cs = k & 7
        row_vec = iota + (128 * j + 16 * cs)
        col_vec = jnp.zeros((16,), jnp.int32) + (8 * i + r)
        vals = plsc.load_gather(g_ref, [row_vec, col_vec])
        tr_ref[i, j, r, pl.ds(16 * cs, 16)] = vals


def _emb_kernel(num_cores, idx_hbm, table_hbm, out_hbm,
                idx_v, g_a, g_b, tr, sg_a, sg_b, ss):
    wid = lax.axis_index("s") * num_cores + lax.axis_index("c")
    b0 = wid * _BSL
    cblk = wid * (_BSL // 128)
    iota = lax.iota(jnp.int32, 16)

    pltpu.sync_copy(idx_hbm.at[:, pl.ds(b0, _BSL)], idx_v)

    def start_gather(t, g, sem):
        pltpu.async_copy(table_hbm.at[idx_v.at[t]], g, sem)

    def wait_gather(g, sem):
        pltpu.make_async_copy(table_hbm.at[idx_v.at[0]], g, sem).wait()

    def wait_store():
        pltpu.make_async_copy(
            tr, out_hbm.at[pl.ds(0, 8), pl.ds(cblk, 4)], ss).wait()

    def step(t, g, sem, first):
        wait_gather(g, sem)
        if not first:
            wait_store()
        _transpose_to_tiles(g, tr, iota)
        pltpu.async_copy(tr, out_hbm.at[pl.ds(8 * t, 8), pl.ds(cblk, 4)], ss)
        start_gather(jnp.minimum(t + 2, _NT - 1), g, sem)

    start_gather(0, g_a, sg_a)
    start_gather(1, g_b, sg_b)
    step(0, g_a, sg_a, True)
    step(1, g_b, sg_b, False)

    def body(k, carry):
        step(2 + 2 * k, g_a, sg_a, False)
        step(3 + 2 * k, g_b, sg_b, False)
        return carry

    lax.fori_loop(0, (_NT - 2) // 2, body, 0)
    wait_store()
    wait_gather(g_a, sg_a)
    wait_gather(g_b, sg_b)


def kernel(input_ids, table):
    B, S = input_ids.shape
    V, D = table.shape
    idx2d = input_ids.T.astype(jnp.int32)

    info = plsc.get_sparse_core_info()
    nw = info.num_cores * info.num_subcores
    assert B == nw * _BSL and S == _NT and D == _D

    mesh = plsc.VectorSubcoreMesh(core_axis_name="c", subcore_axis_name="s")
    emb = functools.partial(
        pl.kernel,
        mesh=mesh,
        out_type=jax.ShapeDtypeStruct((S * 8, B // 128, 8, 128), jnp.float32),
        scratch_types=[
            pltpu.VMEM((_NT, _BSL), jnp.int32),
            pltpu.VMEM((_BSL, _D), jnp.float32),
            pltpu.VMEM((_BSL, _D), jnp.float32),
            pltpu.VMEM((8, _BSL // 128, 8, 128), jnp.float32),
            pltpu.SemaphoreType.DMA,
            pltpu.SemaphoreType.DMA,
            pltpu.SemaphoreType.DMA,
        ],
        compiler_params=pltpu.CompilerParams(use_tc_tiling_on_sc=False,
                                             needs_layout_passes=False),
    )(functools.partial(_emb_kernel, info.num_cores))

    i1 = emb(idx2d, table)
    out = (i1.reshape(S, 8, B // 128, 8, 128)
             .transpose(2, 4, 0, 1, 3)
             .reshape(B, S, D))
    return out


# speedup vs baseline: 1.5376x; 1.1972x over previous
import functools

import jax
import jax.numpy as jnp
from jax import lax
from jax.experimental import pallas as pl
from jax.experimental.pallas import tpu as pltpu
from jax.experimental.pallas import tpu_sc as plsc

_BSL = 512
_NT = 50
_D = 64


def _transpose_to_tiles(g_ref, tr_ref, row_vecs):
    @plsc.parallel_loop(0, 64)
    def body(ir):
        i = ir >> 3
        r = ir & 7
        col_vec = jnp.zeros((16,), jnp.int32) + ir
        for j in range(4):
            for cs in range(8):
                vals = plsc.load_gather(g_ref, [row_vecs[8 * j + cs], col_vec])
                tr_ref[i, j, r, pl.ds(16 * cs, 16)] = vals


def _emb_kernel(num_cores, idx_hbm, table_hbm, out_hbm,
                idx_v, g_a, g_b, tr, sg_a, sg_b, ss):
    wid = lax.axis_index("s") * num_cores + lax.axis_index("c")
    b0 = wid * _BSL
    cblk = wid * (_BSL // 128)
    iota = lax.iota(jnp.int32, 16)
    row_vecs = [iota + (128 * j + 16 * cs)
                for j in range(4) for cs in range(8)]

    pltpu.sync_copy(idx_hbm.at[:, pl.ds(b0, _BSL)], idx_v)

    def start_gather(t, g, sem):
        pltpu.async_copy(table_hbm.at[idx_v.at[t]], g, sem)

    def wait_gather(g, sem):
        pltpu.make_async_copy(table_hbm.at[idx_v.at[0]], g, sem).wait()

    def wait_store():
        pltpu.make_async_copy(
            tr, out_hbm.at[pl.ds(0, 8), pl.ds(cblk, 4)], ss).wait()

    def step(t, g, sem, first):
        wait_gather(g, sem)
        if not first:
            wait_store()
        _transpose_to_tiles(g, tr, row_vecs)
        pltpu.async_copy(tr, out_hbm.at[pl.ds(8 * t, 8), pl.ds(cblk, 4)], ss)
        start_gather(jnp.minimum(t + 2, _NT - 1), g, sem)

    start_gather(0, g_a, sg_a)
    start_gather(1, g_b, sg_b)
    step(0, g_a, sg_a, True)
    step(1, g_b, sg_b, False)

    def body(k, carry):
        step(2 + 2 * k, g_a, sg_a, False)
        step(3 + 2 * k, g_b, sg_b, False)
        return carry

    lax.fori_loop(0, (_NT - 2) // 2, body, 0)
    wait_store()
    wait_gather(g_a, sg_a)
    wait_gather(g_b, sg_b)


def kernel(input_ids, table):
    B, S = input_ids.shape
    V, D = table.shape
    idx2d = input_ids.T.astype(jnp.int32)

    info = plsc.get_sparse_core_info()
    nw = info.num_cores * info.num_subcores
    assert B == nw * _BSL and S == _NT and D == _D

    mesh = plsc.VectorSubcoreMesh(core_axis_name="c", subcore_axis_name="s")
    emb = functools.partial(
        pl.kernel,
        mesh=mesh,
        out_type=jax.ShapeDtypeStruct((S * 8, B // 128, 8, 128), jnp.float32),
        scratch_types=[
            pltpu.VMEM((_NT, _BSL), jnp.int32),
            pltpu.VMEM((_BSL, _D), jnp.float32),
            pltpu.VMEM((_BSL, _D), jnp.float32),
            pltpu.VMEM((8, _BSL // 128, 8, 128), jnp.float32),
            pltpu.SemaphoreType.DMA,
            pltpu.SemaphoreType.DMA,
            pltpu.SemaphoreType.DMA,
        ],
        compiler_params=pltpu.CompilerParams(use_tc_tiling_on_sc=False,
                                             needs_layout_passes=False),
    )(functools.partial(_emb_kernel, info.num_cores))

    i1 = emb(idx2d, table)
    out = (i1.reshape(S, 8, B // 128, 8, 128)
             .transpose(2, 4, 0, 1, 3)
             .reshape(B, S, D))
    return out
